# Initial kernel scaffold; baseline (speedup 1.0000x reference)
import jax
import jax.numpy as jnp
from jax.experimental import pallas as pl


def kernel(x, edge_index, edge_attr, W_node, b_node, ln1_g, ln1_b, W_edge, b_edge, ln2_g, ln2_b, W_msg, b_msg, W_upd, b_upd, ln3_g, ln3_b, proto_n, proto_e, gate_n, gate_e):
    raise NotImplementedError("write your pallas kernel here")


# speedup vs baseline: 1.1282x; 1.1282x over previous
import functools

import jax
import jax.numpy as jnp
from jax import lax
from jax.experimental import pallas as pl
from jax.experimental.pallas import tpu as pltpu
from jax.experimental.pallas import tpu_sc as plsc

B = 16
C = 7
N = 4096
E = 32004
H = 64
EP = 32768
EBLK = 2048
EC = EP // EBLK
NC = 2
NS = 16
NW = NC * NS
GCH = 1024


def _ln(t, g, b):
    mu = jnp.mean(t, axis=-1, keepdims=True)
    d = t - mu
    v = jnp.mean(d * d, axis=-1, keepdims=True)
    return d * lax.rsqrt(v + 1e-5) * g + b


def _gelu(t):
    return 0.5 * t * (1.0 + lax.erf(t * 0.7071067811865476))


def _unit(t):
    n = jnp.sqrt(jnp.sum(t * t, axis=-1, keepdims=True))
    return t / jnp.maximum(n, 1e-6)


def _node_body(x_ref, wn_ref, bn_ref, g1_ref, b1_ref, wt_ref, h_ref, hm_ref):
    t = jnp.dot(x_ref[0], wn_ref[...], preferred_element_type=jnp.float32)
    t = t + bn_ref[...]
    h = _gelu(_ln(t, g1_ref[...], b1_ref[...]))
    h_ref[0] = h
    hm_ref[0] = jnp.dot(h, wt_ref[...], preferred_element_type=jnp.float32)


def _run_node(x_p, wn, bn, g1, b1, wt):
    return pl.pallas_call(
        _node_body,
        grid=(B,),
        in_specs=[
            pl.BlockSpec((1, N, 8), lambda b: (b, 0, 0)),
            pl.BlockSpec((8, H), lambda b: (0, 0)),
            pl.BlockSpec((1, H), lambda b: (0, 0)),
            pl.BlockSpec((1, H), lambda b: (0, 0)),
            pl.BlockSpec((1, H), lambda b: (0, 0)),
            pl.BlockSpec((H, H), lambda b: (0, 0)),
        ],
        out_specs=[
            pl.BlockSpec((1, N, H), lambda b: (b, 0, 0)),
            pl.BlockSpec((1, N, H), lambda b: (b, 0, 0)),
        ],
        out_shape=[
            jax.ShapeDtypeStruct((B, N, H), jnp.float32),
            jax.ShapeDtypeStruct((B, N, H), jnp.float32),
        ],
    )(x_p, wn, bn, g1, b1, wt)


def _sc_mesh():
    return plsc.VectorSubcoreMesh(
        core_axis_name="c", subcore_axis_name="s",
        num_cores=NC, num_subcores=NS)


def _make_gather():
    @functools.partial(
        pl.kernel,
        mesh=_sc_mesh(),
        out_type=jax.ShapeDtypeStruct((B * EP, H), jnp.float32),
        scratch_types=[
            pltpu.VMEM((GCH,), jnp.int32),
            pltpu.VMEM((GCH, H), jnp.float32),
            pltpu.SemaphoreType.DMA,
        ],
        compiler_params=pltpu.CompilerParams(use_tc_tiling_on_sc=False),
    )
    def _gather_k(srcg_hbm, hm_hbm, out_hbm, idx_v, rows_v, sem):
        wid = lax.axis_index("s") * NC + lax.axis_index("c")
        n_chunks = (B * EP) // GCH // NW
        for i in range(n_chunks):
            base = (wid * n_chunks + i) * GCH
            pltpu.sync_copy(srcg_hbm.at[pl.ds(base, GCH)], idx_v)
            pltpu.async_copy(hm_hbm.at[idx_v], rows_v, sem).wait()
            pltpu.sync_copy(rows_v, out_hbm.at[pl.ds(base, GCH)])

    return _gather_k


def _edge_body(ea_ref, hms_ref, pe_ref, gate_ref, we_ref, be_ref, g2_ref,
               b2_ref, wb_ref, bm_ref, m_ref, s1_ref, s0_ref):
    j = pl.program_id(0)
    t = jnp.dot(ea_ref[0], we_ref[...], preferred_element_type=jnp.float32)
    t = t + be_ref[...]
    e = _gelu(_ln(t, g2_ref[...], b2_ref[...]))
    em = jnp.dot(e, wb_ref[...], preferred_element_type=jnp.float32)
    em = em + bm_ref[...]
    m = _gelu(hms_ref[0] + em)
    ridx = lax.broadcasted_iota(jnp.int32, (EBLK, 1), 0) + j * EBLK
    m_ref[0] = jnp.where(ridx < E, m, 0.0)
    en = _unit(e)
    pen = _unit(pe_ref[...])
    sim = jnp.sum(en[None, :, :] * pen, axis=-1)
    ge = jax.nn.sigmoid(gate_ref[...])
    w = jax.nn.sigmoid(sim * 5.0) * ge
    s1_ref[...] = jnp.sum(w * sim, axis=-1).reshape(1, 1, 1, C)
    s0_ref[...] = jnp.sum(w, axis=-1).reshape(1, 1, 1, C)


def _run_edge(ea_p, hms, pe_p, gate_p, we, be, g2, b2, wb, bm):
    return pl.pallas_call(
        _edge_body,
        grid=(EC, B),
        in_specs=[
            pl.BlockSpec((1, EBLK, 8), lambda j, b: (b, j, 0)),
            pl.BlockSpec((1, EBLK, H), lambda j, b: (b, j, 0)),
            pl.BlockSpec((C, EBLK, H), lambda j, b: (0, j, 0)),
            pl.BlockSpec((C, EBLK), lambda j, b: (0, j)),
            pl.BlockSpec((8, H), lambda j, b: (0, 0)),
            pl.BlockSpec((1, H), lambda j, b: (0, 0)),
            pl.BlockSpec((1, H), lambda j, b: (0, 0)),
            pl.BlockSpec((1, H), lambda j, b: (0, 0)),
            pl.BlockSpec((H, H), lambda j, b: (0, 0)),
            pl.BlockSpec((1, H), lambda j, b: (0, 0)),
        ],
        out_specs=[
            pl.BlockSpec((1, EBLK, H), lambda j, b: (b, j, 0)),
            pl.BlockSpec((1, 1, 1, C), lambda j, b: (j, b, 0, 0)),
            pl.BlockSpec((1, 1, 1, C), lambda j, b: (j, b, 0, 0)),
        ],
        out_shape=[
            jax.ShapeDtypeStruct((B, EP, H), jnp.float32),
            jax.ShapeDtypeStruct((EC, B, 1, C), jnp.float32),
            jax.ShapeDtypeStruct((EC, B, 1, C), jnp.float32),
        ],
    )(ea_p, hms, pe_p, gate_p, we, be, g2, b2, wb, bm)


EPT = EP // NS
BPC = B // NC
NPT = N // NS


def _make_scatter():
    @functools.partial(
        pl.kernel,
        mesh=_sc_mesh(),
        out_type=jax.ShapeDtypeStruct((B * N, H), jnp.float32),
        scratch_types=[
            pltpu.VMEM((GCH,), jnp.int32),
            pltpu.VMEM((GCH,), jnp.int32),
            pltpu.VMEM((GCH, H), jnp.float32),
            pltpu.VMEM_SHARED((N, H), jnp.float32),
        ],
        compiler_params=pltpu.CompilerParams(use_tc_tiling_on_sc=False),
    )
    def _scatter_k(m_hbm, dst_hbm, zeros_hbm, out_hbm, ib0, ib1, mb, agg_s):
        c = lax.axis_index("c")
        s = lax.axis_index("s")
        pltpu.sync_copy(dst_hbm.at[pl.ds(s * EPT, GCH)], ib0)
        pltpu.sync_copy(dst_hbm.at[pl.ds(s * EPT + GCH, GCH)], ib1)
        for t in range(BPC):
            b = c * BPC + t
            pltpu.sync_copy(zeros_hbm.at[pl.ds(s * NPT, NPT)],
                            agg_s.at[pl.ds(s * NPT, NPT)])
            plsc.subcore_barrier()
            pltpu.sync_copy(m_hbm.at[pl.ds(b * EP + s * EPT, GCH)], mb)
            pltpu.sync_copy(mb, agg_s.at[ib0], add=True)
            pltpu.sync_copy(m_hbm.at[pl.ds(b * EP + s * EPT + GCH, GCH)], mb)
            pltpu.sync_copy(mb, agg_s.at[ib1], add=True)
            plsc.subcore_barrier()
            pltpu.sync_copy(agg_s.at[pl.ds(s * NPT, NPT)],
                            out_hbm.at[pl.ds(b * N + s * NPT, NPT)])
            plsc.subcore_barrier()

    return _scatter_k


def _final_body(h_ref, agg_ref, pn_ref, gn_ref, wu_ref, bu_ref, g3_ref,
                b3_ref, s1_ref, s0_ref, out_ref):
    a = jnp.dot(agg_ref[0], wu_ref[...], preferred_element_type=jnp.float32)
    a = a + bu_ref[...]
    h2 = _ln(h_ref[0] + a, g3_ref[...], b3_ref[...])
    hn = _unit(h2)
    pnn = _unit(pn_ref[...])
    sim = jnp.sum(hn[None, :, :] * pnn, axis=-1)
    gn = jax.nn.sigmoid(gn_ref[...])
    w = jax.nn.sigmoid(sim * 5.0) * gn
    ns = jnp.sum(w * sim, axis=-1) / jnp.maximum(jnp.sum(w, axis=-1), 1e-6)
    es1 = jnp.sum(s1_ref[...], axis=(0, 1, 2))
    es0 = jnp.sum(s0_ref[...], axis=(0, 1, 2))
    es = es1 / jnp.maximum(es0, 1e-6)
    out_ref[...] = (ns + 0.5 * es).reshape(1, 1, C)


def _run_final(h, agg, pn, gn, wu, bu, g3, b3, s1, s0):
    return pl.pallas_call(
        _final_body,
        grid=(B,),
        in_specs=[
            pl.BlockSpec((1, N, H), lambda b: (b, 0, 0)),
            pl.BlockSpec((1, N, H), lambda b: (b, 0, 0)),
            pl.BlockSpec((C, N, H), lambda b: (0, 0, 0)),
            pl.BlockSpec((C, N), lambda b: (0, 0)),
            pl.BlockSpec((H, H), lambda b: (0, 0)),
            pl.BlockSpec((1, H), lambda b: (0, 0)),
            pl.BlockSpec((1, H), lambda b: (0, 0)),
            pl.BlockSpec((1, H), lambda b: (0, 0)),
            pl.BlockSpec((EC, 1, 1, C), lambda b: (0, b, 0, 0)),
            pl.BlockSpec((EC, 1, 1, C), lambda b: (0, b, 0, 0)),
        ],
        out_specs=pl.BlockSpec((1, 1, C), lambda b: (b, 0, 0)),
        out_shape=jax.ShapeDtypeStruct((B, 1, C), jnp.float32),
    )(h, agg, pn, gn, wu, bu, g3, b3, s1, s0)


def kernel(x, edge_index, edge_attr, W_node, b_node, ln1_g, ln1_b, W_edge,
           b_edge, ln2_g, ln2_b, W_msg, b_msg, W_upd, b_upd, ln3_g, ln3_b,
           proto_n, proto_e, gate_n, gate_e):
    f32 = jnp.float32
    x_p = jnp.pad(x, ((0, 0), (0, 0), (0, 1)))
    wn = jnp.pad(W_node, ((0, 1), (0, 0)))
    row = lambda v: v.reshape(1, H)
    h, hm = _run_node(x_p, wn, row(b_node), row(ln1_g), row(ln1_b),
                      W_msg[:H, :])

    src = edge_index[0]
    dst = edge_index[1]
    srcg = (jnp.pad(src, (0, EP - E))[None, :]
            + (jnp.arange(B, dtype=jnp.int32) * N)[:, None]).reshape(-1)
    hms = _make_gather()(srcg, hm.reshape(B * N, H)).reshape(B, EP, H)

    ea_p = jnp.pad(edge_attr, ((0, 0), (0, EP - E), (0, 3)))
    pe_p = jnp.pad(proto_e, ((0, 0), (0, EP - E), (0, 0)))
    gate_p = jnp.pad(gate_e, ((0, 0), (0, EP - E)), constant_values=-1e30)
    we = jnp.pad(W_edge, ((0, 3), (0, 0)))
    m, s1, s0 = _run_edge(ea_p, hms, pe_p, gate_p, we, row(b_edge),
                          row(ln2_g), row(ln2_b), W_msg[H:, :], row(b_msg))

    dst_p = jnp.pad(dst, (0, EP - E))
    zeros = jnp.zeros((N, H), f32)
    agg2 = _make_scatter()(m.reshape(B * EP, H), dst_p, zeros)
    agg = agg2.reshape(B, N, H)

    logits = _run_final(h, agg, proto_n, gate_n, W_upd, row(b_upd),
                        row(ln3_g), row(ln3_b), s1, s0)
    return logits.reshape(B, C)


# speedup vs baseline: 3.0368x; 2.6917x over previous
import functools

import jax
import jax.numpy as jnp
from jax import lax
from jax.experimental import pallas as pl
from jax.experimental.pallas import tpu as pltpu
from jax.experimental.pallas import tpu_sc as plsc

B = 16
C = 7
N = 4096
E = 32004
H = 64
EP = 32768
EBLK = 2048
EC = EP // EBLK
NC = 2
NS = 16
NW = NC * NS
GCH = 1024


def _ln(t, g, b):
    mu = jnp.mean(t, axis=-1, keepdims=True)
    d = t - mu
    v = jnp.mean(d * d, axis=-1, keepdims=True)
    return d * lax.rsqrt(v + 1e-5) * g + b


def _gelu(t):
    return 0.5 * t * (1.0 + lax.erf(t * 0.7071067811865476))


def _unit(t):
    n = jnp.sqrt(jnp.sum(t * t, axis=-1, keepdims=True))
    return t / jnp.maximum(n, 1e-6)


def _node_body(x_ref, wn_ref, bn_ref, g1_ref, b1_ref, wt_ref, h_ref, hm_ref):
    t = jnp.dot(x_ref[0], wn_ref[...], preferred_element_type=jnp.float32)
    t = t + bn_ref[...]
    h = _gelu(_ln(t, g1_ref[...], b1_ref[...]))
    h_ref[0] = h
    hm_ref[0] = jnp.dot(h, wt_ref[...], preferred_element_type=jnp.float32)


def _run_node(x_p, wn, bn, g1, b1, wt):
    return pl.pallas_call(
        _node_body,
        grid=(B,),
        in_specs=[
            pl.BlockSpec((1, N, 7), lambda b: (b, 0, 0)),
            pl.BlockSpec((7, H), lambda b: (0, 0)),
            pl.BlockSpec((1, H), lambda b: (0, 0)),
            pl.BlockSpec((1, H), lambda b: (0, 0)),
            pl.BlockSpec((1, H), lambda b: (0, 0)),
            pl.BlockSpec((H, H), lambda b: (0, 0)),
        ],
        out_specs=[
            pl.BlockSpec((1, N, H), lambda b: (b, 0, 0)),
            pl.BlockSpec((1, N, H), lambda b: (b, 0, 0)),
        ],
        out_shape=[
            jax.ShapeDtypeStruct((B, N, H), jnp.float32),
            jax.ShapeDtypeStruct((B, N, H), jnp.float32),
        ],
    )(x_p, wn, bn, g1, b1, wt)


def _sc_mesh():
    return plsc.VectorSubcoreMesh(
        core_axis_name="c", subcore_axis_name="s",
        num_cores=NC, num_subcores=NS)


def _make_gather():
    @functools.partial(
        pl.kernel,
        mesh=_sc_mesh(),
        out_type=jax.ShapeDtypeStruct((B * EP, H), jnp.float32),
        scratch_types=[
            pltpu.VMEM((GCH,), jnp.int32),
            pltpu.VMEM((GCH, H), jnp.float32),
            pltpu.SemaphoreType.DMA,
        ],
        compiler_params=pltpu.CompilerParams(use_tc_tiling_on_sc=False),
    )
    def _gather_k(srcg_hbm, hm_hbm, out_hbm, idx_v, rows_v, sem):
        wid = lax.axis_index("s") * NC + lax.axis_index("c")
        n_chunks = (B * EP) // GCH // NW
        for i in range(n_chunks):
            base = (wid * n_chunks + i) * GCH
            pltpu.sync_copy(srcg_hbm.at[pl.ds(base, GCH)], idx_v)
            pltpu.async_copy(hm_hbm.at[idx_v], rows_v, sem).wait()
            pltpu.sync_copy(rows_v, out_hbm.at[pl.ds(base, GCH)])

    return _gather_k


def _edge_body(ea_ref, hms_ref, pe_ref, ge_ref, b7_ref, we_ref, be_ref,
               g2_ref, b2_ref, wb_ref, bm_ref, m_ref, s1_ref, s0_ref,
               pen_s, ge_s):
    j = pl.program_id(0)
    ones64 = jnp.ones((H, H), jnp.bfloat16)
    ridx = lax.broadcasted_iota(jnp.int32, (EBLK, 1), 0) + j * EBLK
    rowmask = ridx < E

    @pl.when(pl.program_id(1) == 0)
    def _():
        for c in range(C):
            pc = pe_ref[c]
            ss = jnp.dot((pc * pc).astype(jnp.bfloat16), ones64,
                         preferred_element_type=jnp.float32)
            pen_s[c] = (pc * lax.rsqrt(jnp.maximum(ss, 1e-12))
                        ).astype(jnp.bfloat16)
        ge_s[...] = jnp.where(rowmask, jax.nn.sigmoid(ge_ref[...]), 0.0)

    t = jnp.dot(ea_ref[0], we_ref[...], preferred_element_type=jnp.float32)
    t = t + be_ref[...]
    e = _gelu(_ln(t, g2_ref[...], b2_ref[...]))
    em = jnp.dot(e, wb_ref[...], preferred_element_type=jnp.float32)
    em = em + bm_ref[...]
    m = _gelu(hms_ref[0] + em)
    m_ref[0] = jnp.where(rowmask, m, 0.0)

    ss = jnp.dot((e * e).astype(jnp.bfloat16), ones64,
                 preferred_element_type=jnp.float32)
    en = (e * lax.rsqrt(jnp.maximum(ss, 1e-12))).astype(jnp.bfloat16)
    u = jnp.concatenate([en * pen_s[c] for c in range(C)], axis=-1)
    sim = jnp.dot(u, b7_ref[...], preferred_element_type=jnp.float32)
    sim = jnp.where(rowmask, sim, 0.0)
    w = jax.nn.sigmoid(sim * 5.0) * ge_s[...]
    s1_ref[...] = jnp.sum(w * sim, axis=0).reshape(1, 1, 1, C)
    s0_ref[...] = jnp.sum(w, axis=0).reshape(1, 1, 1, C)


def _run_edge(ea, hms, pe, ge_t, b7, we, be, g2, b2, wb, bm):
    return pl.pallas_call(
        _edge_body,
        grid=(EC, B),
        in_specs=[
            pl.BlockSpec((1, EBLK, 5), lambda j, b: (b, j, 0)),
            pl.BlockSpec((1, EBLK, H), lambda j, b: (b, j, 0)),
            pl.BlockSpec((C, EBLK, H), lambda j, b: (0, j, 0)),
            pl.BlockSpec((EBLK, C), lambda j, b: (j, 0)),
            pl.BlockSpec((C * H, C), lambda j, b: (0, 0)),
            pl.BlockSpec((5, H), lambda j, b: (0, 0)),
            pl.BlockSpec((1, H), lambda j, b: (0, 0)),
            pl.BlockSpec((1, H), lambda j, b: (0, 0)),
            pl.BlockSpec((1, H), lambda j, b: (0, 0)),
            pl.BlockSpec((H, H), lambda j, b: (0, 0)),
            pl.BlockSpec((1, H), lambda j, b: (0, 0)),
        ],
        out_specs=[
            pl.BlockSpec((1, EBLK, H), lambda j, b: (b, j, 0)),
            pl.BlockSpec((1, 1, 1, C), lambda j, b: (j, b, 0, 0)),
            pl.BlockSpec((1, 1, 1, C), lambda j, b: (j, b, 0, 0)),
        ],
        out_shape=[
            jax.ShapeDtypeStruct((B, EP, H), jnp.float32),
            jax.ShapeDtypeStruct((EC, B, 1, C), jnp.float32),
            jax.ShapeDtypeStruct((EC, B, 1, C), jnp.float32),
        ],
        scratch_shapes=[pltpu.VMEM((C, EBLK, H), jnp.bfloat16),
                        pltpu.VMEM((EBLK, C), jnp.float32)],
    )(ea, hms, pe, ge_t, b7, we, be, g2, b2, wb, bm)


EPT = EP // NS
BPC = B // NC
NPT = N // NS


def _make_scatter():
    @functools.partial(
        pl.kernel,
        mesh=_sc_mesh(),
        out_type=jax.ShapeDtypeStruct((B * N, H), jnp.float32),
        scratch_types=[
            pltpu.VMEM((GCH,), jnp.int32),
            pltpu.VMEM((GCH,), jnp.int32),
            pltpu.VMEM((GCH, H), jnp.float32),
            pltpu.VMEM_SHARED((N, H), jnp.float32),
        ],
        compiler_params=pltpu.CompilerParams(use_tc_tiling_on_sc=False),
    )
    def _scatter_k(m_hbm, dst_hbm, zeros_hbm, out_hbm, ib0, ib1, mb, agg_s):
        c = lax.axis_index("c")
        s = lax.axis_index("s")
        pltpu.sync_copy(dst_hbm.at[pl.ds(s * EPT, GCH)], ib0)
        pltpu.sync_copy(dst_hbm.at[pl.ds(s * EPT + GCH, GCH)], ib1)
        for t in range(BPC):
            b = c * BPC + t
            pltpu.sync_copy(zeros_hbm.at[pl.ds(s * NPT, NPT)],
                            agg_s.at[pl.ds(s * NPT, NPT)])
            plsc.subcore_barrier()
            pltpu.sync_copy(m_hbm.at[pl.ds(b * EP + s * EPT, GCH)], mb)
            pltpu.sync_copy(mb, agg_s.at[ib0], add=True)
            pltpu.sync_copy(m_hbm.at[pl.ds(b * EP + s * EPT + GCH, GCH)], mb)
            pltpu.sync_copy(mb, agg_s.at[ib1], add=True)
            plsc.subcore_barrier()
            pltpu.sync_copy(agg_s.at[pl.ds(s * NPT, NPT)],
                            out_hbm.at[pl.ds(b * N + s * NPT, NPT)])
            plsc.subcore_barrier()

    return _scatter_k


def _final_body(h_ref, agg_ref, pn_ref, gn_ref, b7_ref, wu_ref, bu_ref,
                g3_ref, b3_ref, s1_ref, s0_ref, out_ref, pnn_s, gn_s):
    ones64 = jnp.ones((H, H), jnp.bfloat16)

    @pl.when(pl.program_id(0) == 0)
    def _():
        for c in range(C):
            pc = pn_ref[c]
            ss = jnp.dot((pc * pc).astype(jnp.bfloat16), ones64,
                         preferred_element_type=jnp.float32)
            pnn_s[c] = (pc * lax.rsqrt(jnp.maximum(ss, 1e-12))
                        ).astype(jnp.bfloat16)
        gn_s[...] = jax.nn.sigmoid(gn_ref[...])

    a = jnp.dot(agg_ref[0], wu_ref[...], preferred_element_type=jnp.float32)
    a = a + bu_ref[...]
    h2 = _ln(h_ref[0] + a, g3_ref[...], b3_ref[...])
    ss = jnp.dot((h2 * h2).astype(jnp.bfloat16), ones64,
                 preferred_element_type=jnp.float32)
    hn = (h2 * lax.rsqrt(jnp.maximum(ss, 1e-12))).astype(jnp.bfloat16)
    u = jnp.concatenate([hn * pnn_s[c] for c in range(C)], axis=-1)
    sim = jnp.dot(u, b7_ref[...], preferred_element_type=jnp.float32)
    w = jax.nn.sigmoid(sim * 5.0) * gn_s[...]
    ns = jnp.sum(w * sim, axis=0) / jnp.maximum(jnp.sum(w, axis=0), 1e-6)
    es1 = jnp.sum(s1_ref[...], axis=(0, 1, 2))
    es0 = jnp.sum(s0_ref[...], axis=(0, 1, 2))
    es = es1 / jnp.maximum(es0, 1e-6)
    out_ref[...] = (ns + 0.5 * es).reshape(1, 1, C)


def _run_final(h, agg, pn, gn_t, b7, wu, bu, g3, b3, s1, s0):
    return pl.pallas_call(
        _final_body,
        grid=(B,),
        in_specs=[
            pl.BlockSpec((1, N, H), lambda b: (b, 0, 0)),
            pl.BlockSpec((1, N, H), lambda b: (b, 0, 0)),
            pl.BlockSpec((C, N, H), lambda b: (0, 0, 0)),
            pl.BlockSpec((N, C), lambda b: (0, 0)),
            pl.BlockSpec((C * H, C), lambda b: (0, 0)),
            pl.BlockSpec((H, H), lambda b: (0, 0)),
            pl.BlockSpec((1, H), lambda b: (0, 0)),
            pl.BlockSpec((1, H), lambda b: (0, 0)),
            pl.BlockSpec((1, H), lambda b: (0, 0)),
            pl.BlockSpec((EC, 1, 1, C), lambda b: (0, b, 0, 0)),
            pl.BlockSpec((EC, 1, 1, C), lambda b: (0, b, 0, 0)),
        ],
        out_specs=pl.BlockSpec((1, 1, C), lambda b: (b, 0, 0)),
        out_shape=jax.ShapeDtypeStruct((B, 1, C), jnp.float32),
        scratch_shapes=[pltpu.VMEM((C, N, H), jnp.bfloat16),
                        pltpu.VMEM((N, C), jnp.float32)],
    )(h, agg, pn, gn_t, b7, wu, bu, g3, b3, s1, s0)


def kernel(x, edge_index, edge_attr, W_node, b_node, ln1_g, ln1_b, W_edge,
           b_edge, ln2_g, ln2_b, W_msg, b_msg, W_upd, b_upd, ln3_g, ln3_b,
           proto_n, proto_e, gate_n, gate_e):
    f32 = jnp.float32
    row = lambda v: v.reshape(1, H)
    h, hm = _run_node(x, W_node, row(b_node), row(ln1_g), row(ln1_b),
                      W_msg[:H, :])

    src = edge_index[0]
    dst = edge_index[1]
    srcg = (jnp.pad(src, (0, EP - E))[None, :]
            + (jnp.arange(B, dtype=jnp.int32) * N)[:, None]).reshape(-1)
    hms = _make_gather()(srcg, hm.reshape(B * N, H)).reshape(B, EP, H)

    b7 = jnp.repeat(jnp.eye(C, dtype=jnp.bfloat16), H, axis=0)
    m, s1, s0 = _run_edge(edge_attr, hms, proto_e, gate_e.T, b7, W_edge,
                          row(b_edge), row(ln2_g), row(ln2_b), W_msg[H:, :],
                          row(b_msg))

    dst_p = jnp.pad(dst, (0, EP - E))
    zeros = jnp.zeros((N, H), f32)
    agg2 = _make_scatter()(m.reshape(B * EP, H), dst_p, zeros)
    agg = agg2.reshape(B, N, H)

    logits = _run_final(h, agg, proto_n, gate_n.T, b7, W_upd, row(b_upd),
                        row(ln3_g), row(ln3_b), s1, s0)
    return logits.reshape(B, C)


# speedup vs baseline: 3.1487x; 1.0368x over previous
import functools

import jax
import jax.numpy as jnp
from jax import lax
from jax.experimental import pallas as pl
from jax.experimental.pallas import tpu as pltpu
from jax.experimental.pallas import tpu_sc as plsc

B = 16
C = 7
N = 4096
E = 32004
H = 64
EP = 32768
EBLK = 4096
EC = EP // EBLK
NC = 2
NS = 16
NW = NC * NS
GCH = 512


def _ln(t, g, b):
    mu = jnp.mean(t, axis=-1, keepdims=True)
    d = t - mu
    v = jnp.mean(d * d, axis=-1, keepdims=True)
    return d * lax.rsqrt(v + 1e-5) * g + b


def _gelu(t):
    return 0.5 * t * (1.0 + lax.erf(t * 0.7071067811865476))


def _unit(t):
    n = jnp.sqrt(jnp.sum(t * t, axis=-1, keepdims=True))
    return t / jnp.maximum(n, 1e-6)


def _node_body(x_ref, wn_ref, bn_ref, g1_ref, b1_ref, wt_ref, h_ref, hm_ref):
    t = jnp.dot(x_ref[0], wn_ref[...], preferred_element_type=jnp.float32)
    t = t + bn_ref[...]
    h = _gelu(_ln(t, g1_ref[...], b1_ref[...]))
    h_ref[0] = h
    hm_ref[0] = jnp.dot(h, wt_ref[...], preferred_element_type=jnp.float32)


def _run_node(x_p, wn, bn, g1, b1, wt):
    return pl.pallas_call(
        _node_body,
        grid=(B,),
        in_specs=[
            pl.BlockSpec((1, N, 7), lambda b: (b, 0, 0)),
            pl.BlockSpec((7, H), lambda b: (0, 0)),
            pl.BlockSpec((1, H), lambda b: (0, 0)),
            pl.BlockSpec((1, H), lambda b: (0, 0)),
            pl.BlockSpec((1, H), lambda b: (0, 0)),
            pl.BlockSpec((H, H), lambda b: (0, 0)),
        ],
        out_specs=[
            pl.BlockSpec((1, N, H), lambda b: (b, 0, 0)),
            pl.BlockSpec((1, N, H), lambda b: (b, 0, 0)),
        ],
        out_shape=[
            jax.ShapeDtypeStruct((B, N, H), jnp.float32),
            jax.ShapeDtypeStruct((B, N, H), jnp.float32),
        ],
    )(x_p, wn, bn, g1, b1, wt)


def _sc_mesh():
    return plsc.VectorSubcoreMesh(
        core_axis_name="c", subcore_axis_name="s",
        num_cores=NC, num_subcores=NS)


def _make_gather():
    @functools.partial(
        pl.kernel,
        mesh=_sc_mesh(),
        out_type=jax.ShapeDtypeStruct((B * EP, H), jnp.float32),
        scratch_types=[
            pltpu.VMEM((GCH,), jnp.int32),
            pltpu.VMEM((GCH,), jnp.int32),
            pltpu.VMEM((GCH, H), jnp.float32),
            pltpu.VMEM((GCH, H), jnp.float32),
            pltpu.SemaphoreType.DMA,
            pltpu.SemaphoreType.DMA,
            pltpu.SemaphoreType.DMA,
            pltpu.SemaphoreType.DMA,
        ],
        compiler_params=pltpu.CompilerParams(use_tc_tiling_on_sc=False),
    )
    def _gather_k(srcg_hbm, hm_hbm, out_hbm, i0, i1, r0, r1, gs0, gs1,
                  ws0, ws1):
        wid = lax.axis_index("s") * NC + lax.axis_index("c")
        n_chunks = (B * EP) // GCH // NW
        idx = [i0, i1]
        rows = [r0, r1]
        gs = [gs0, gs1]
        ws = [ws0, ws1]
        base = lambda i: (wid * n_chunks + i) * GCH
        pltpu.sync_copy(srcg_hbm.at[pl.ds(base(0), GCH)], i0)
        g = {0: pltpu.async_copy(hm_hbm.at[i0], r0, gs0)}
        w = {}
        for i in range(n_chunks):
            cur = i & 1
            nxt = 1 - cur
            if i + 1 < n_chunks:
                pltpu.sync_copy(srcg_hbm.at[pl.ds(base(i + 1), GCH)],
                                idx[nxt])
                if i >= 1:
                    w[i - 1].wait()
                g[i + 1] = pltpu.async_copy(hm_hbm.at[idx[nxt]], rows[nxt],
                                            gs[nxt])
            g[i].wait()
            w[i] = pltpu.async_copy(rows[cur],
                                    out_hbm.at[pl.ds(base(i), GCH)], ws[cur])
        w[n_chunks - 2].wait()
        w[n_chunks - 1].wait()

    return _gather_k


def _prep_body(pe_ref, gt_ref, pen_ref, ges_ref):
    j = pl.program_id(0)
    ones64 = jnp.ones((H, H), jnp.bfloat16)
    ridx = lax.broadcasted_iota(jnp.int32, (EBLK, 1), 0) + j * EBLK
    rowmask = ridx < E
    for c in range(C):
        pc = jnp.where(rowmask, pe_ref[c], 0.0)
        ss = jnp.dot((pc * pc).astype(jnp.bfloat16), ones64,
                     preferred_element_type=jnp.float32)
        pen_ref[c] = (pc * lax.rsqrt(jnp.maximum(ss, 1e-12))
                      ).astype(jnp.bfloat16)
    ges_ref[...] = jnp.where(rowmask, jax.nn.sigmoid(gt_ref[...]), 0.0)


def _run_prep(pe, gt):
    return pl.pallas_call(
        _prep_body,
        grid=(EC,),
        in_specs=[
            pl.BlockSpec((C, EBLK, H), lambda j: (0, j, 0)),
            pl.BlockSpec((EBLK, C), lambda j: (j, 0)),
        ],
        out_specs=[
            pl.BlockSpec((C, EBLK, H), lambda j: (0, j, 0)),
            pl.BlockSpec((EBLK, C), lambda j: (j, 0)),
        ],
        out_shape=[
            jax.ShapeDtypeStruct((C, EP, H), jnp.bfloat16),
            jax.ShapeDtypeStruct((EP, C), jnp.float32),
        ],
    )(pe, gt)


def _edge_body(ea_ref, hms_ref, pen_ref, ges_ref, b7_ref, we_ref, be_ref,
               g2_ref, b2_ref, wb_ref, bm_ref, m_ref, s1_ref, s0_ref):
    j = pl.program_id(0)
    ones64 = jnp.ones((H, H), jnp.bfloat16)
    ridx = lax.broadcasted_iota(jnp.int32, (EBLK, 1), 0) + j * EBLK
    rowmask = ridx < E

    t = jnp.dot(ea_ref[0], we_ref[...], preferred_element_type=jnp.float32)
    t = t + be_ref[...]
    e = _gelu(_ln(t, g2_ref[...], b2_ref[...]))
    em = jnp.dot(e, wb_ref[...], preferred_element_type=jnp.float32)
    em = em + bm_ref[...]
    m = _gelu(hms_ref[0] + em)
    m_ref[0] = jnp.where(rowmask, m, 0.0)

    ss = jnp.dot((e * e).astype(jnp.bfloat16), ones64,
                 preferred_element_type=jnp.float32)
    en = (e * lax.rsqrt(jnp.maximum(ss, 1e-12))).astype(jnp.bfloat16)
    u = jnp.concatenate([en * pen_ref[c] for c in range(C)], axis=-1)
    sim = jnp.dot(u, b7_ref[...], preferred_element_type=jnp.float32)
    sim = jnp.where(rowmask, sim, 0.0)
    w = jax.nn.sigmoid(sim * 5.0) * ges_ref[...]
    s1_ref[...] = jnp.sum(w * sim, axis=0).reshape(1, 1, 1, C)
    s0_ref[...] = jnp.sum(w, axis=0).reshape(1, 1, 1, C)


def _run_edge(ea, hms, pen, ges, b7, we, be, g2, b2, wb, bm):
    return pl.pallas_call(
        _edge_body,
        grid=(EC, B),
        in_specs=[
            pl.BlockSpec((1, EBLK, 5), lambda j, b: (b, j, 0)),
            pl.BlockSpec((1, EBLK, H), lambda j, b: (b, j, 0)),
            pl.BlockSpec((C, EBLK, H), lambda j, b: (0, j, 0)),
            pl.BlockSpec((EBLK, C), lambda j, b: (j, 0)),
            pl.BlockSpec((C * H, C), lambda j, b: (0, 0)),
            pl.BlockSpec((5, H), lambda j, b: (0, 0)),
            pl.BlockSpec((1, H), lambda j, b: (0, 0)),
            pl.BlockSpec((1, H), lambda j, b: (0, 0)),
            pl.BlockSpec((1, H), lambda j, b: (0, 0)),
            pl.BlockSpec((H, H), lambda j, b: (0, 0)),
            pl.BlockSpec((1, H), lambda j, b: (0, 0)),
        ],
        out_specs=[
            pl.BlockSpec((1, EBLK, H), lambda j, b: (b, j, 0)),
            pl.BlockSpec((1, 1, 1, C), lambda j, b: (j, b, 0, 0)),
            pl.BlockSpec((1, 1, 1, C), lambda j, b: (j, b, 0, 0)),
        ],
        out_shape=[
            jax.ShapeDtypeStruct((B, EP, H), jnp.float32),
            jax.ShapeDtypeStruct((EC, B, 1, C), jnp.float32),
            jax.ShapeDtypeStruct((EC, B, 1, C), jnp.float32),
        ],
    )(ea, hms, pen, ges, b7, we, be, g2, b2, wb, bm)


EPT = EP // NS
BPC = B // NC
NPT = N // NS


SCH = 512
NSCH = EPT // SCH


def _make_scatter():
    @functools.partial(
        pl.kernel,
        mesh=_sc_mesh(),
        out_type=jax.ShapeDtypeStruct((B * N, H), jnp.float32),
        scratch_types=[
            pltpu.VMEM((SCH,), jnp.int32),
            pltpu.VMEM((SCH,), jnp.int32),
            pltpu.VMEM((SCH,), jnp.int32),
            pltpu.VMEM((SCH,), jnp.int32),
            pltpu.VMEM((SCH, H), jnp.float32),
            pltpu.VMEM((SCH, H), jnp.float32),
            pltpu.VMEM_SHARED((N, H), jnp.float32),
            pltpu.SemaphoreType.DMA,
            pltpu.SemaphoreType.DMA,
            pltpu.SemaphoreType.DMA,
            pltpu.SemaphoreType.DMA,
        ],
        compiler_params=pltpu.CompilerParams(use_tc_tiling_on_sc=False),
    )
    def _scatter_k(m_hbm, dst_hbm, zeros_hbm, out_hbm, ib0, ib1, ib2, ib3,
                   mb0, mb1, agg_s, ls0, ls1, ss0, ss1):
        c = lax.axis_index("c")
        s = lax.axis_index("s")
        ibs = [ib0, ib1, ib2, ib3]
        mb = [mb0, mb1]
        ls = [ls0, ls1]
        ssem = [ss0, ss1]
        for k in range(NSCH):
            pltpu.sync_copy(dst_hbm.at[pl.ds(s * EPT + k * SCH, SCH)], ibs[k])
        for t in range(BPC):
            b = c * BPC + t
            pltpu.sync_copy(zeros_hbm.at[pl.ds(s * NPT, NPT)],
                            agg_s.at[pl.ds(s * NPT, NPT)])
            plsc.subcore_barrier()
            mchunk = lambda k: m_hbm.at[
                pl.ds(b * EP + s * EPT + k * SCH, SCH)]
            ld = {0: pltpu.async_copy(mchunk(0), mb0, ls0)}
            sc = {}
            for k in range(NSCH):
                p = k & 1
                q = 1 - p
                if k + 1 < NSCH:
                    if k >= 1:
                        sc[k - 1].wait()
                    ld[k + 1] = pltpu.async_copy(mchunk(k + 1), mb[q], ls[q])
                ld[k].wait()
                sc[k] = pltpu.async_copy(mb[p], agg_s.at[ibs[k]], ssem[p],
                                         add=True)
            sc[NSCH - 2].wait()
            sc[NSCH - 1].wait()
            plsc.subcore_barrier()
            pltpu.sync_copy(agg_s.at[pl.ds(s * NPT, NPT)],
                            out_hbm.at[pl.ds(b * N + s * NPT, NPT)])
            plsc.subcore_barrier()

    return _scatter_k


def _final_body(h_ref, agg_ref, pn_ref, gn_ref, b7_ref, wu_ref, bu_ref,
                g3_ref, b3_ref, s1_ref, s0_ref, out_ref, pnn_s, gn_s):
    ones64 = jnp.ones((H, H), jnp.bfloat16)

    @pl.when(pl.program_id(0) == 0)
    def _():
        for c in range(C):
            pc = pn_ref[c]
            ss = jnp.dot((pc * pc).astype(jnp.bfloat16), ones64,
                         preferred_element_type=jnp.float32)
            pnn_s[c] = (pc * lax.rsqrt(jnp.maximum(ss, 1e-12))
                        ).astype(jnp.bfloat16)
        gn_s[...] = jax.nn.sigmoid(gn_ref[...])

    a = jnp.dot(agg_ref[0], wu_ref[...], preferred_element_type=jnp.float32)
    a = a + bu_ref[...]
    h2 = _ln(h_ref[0] + a, g3_ref[...], b3_ref[...])
    ss = jnp.dot((h2 * h2).astype(jnp.bfloat16), ones64,
                 preferred_element_type=jnp.float32)
    hn = (h2 * lax.rsqrt(jnp.maximum(ss, 1e-12))).astype(jnp.bfloat16)
    u = jnp.concatenate([hn * pnn_s[c] for c in range(C)], axis=-1)
    sim = jnp.dot(u, b7_ref[...], preferred_element_type=jnp.float32)
    w = jax.nn.sigmoid(sim * 5.0) * gn_s[...]
    ns = jnp.sum(w * sim, axis=0) / jnp.maximum(jnp.sum(w, axis=0), 1e-6)
    es1 = jnp.sum(s1_ref[...], axis=(0, 1, 2))
    es0 = jnp.sum(s0_ref[...], axis=(0, 1, 2))
    es = es1 / jnp.maximum(es0, 1e-6)
    out_ref[...] = (ns + 0.5 * es).reshape(1, 1, C)


def _run_final(h, agg, pn, gn_t, b7, wu, bu, g3, b3, s1, s0):
    return pl.pallas_call(
        _final_body,
        grid=(B,),
        in_specs=[
            pl.BlockSpec((1, N, H), lambda b: (b, 0, 0)),
            pl.BlockSpec((1, N, H), lambda b: (b, 0, 0)),
            pl.BlockSpec((C, N, H), lambda b: (0, 0, 0)),
            pl.BlockSpec((N, C), lambda b: (0, 0)),
            pl.BlockSpec((C * H, C), lambda b: (0, 0)),
            pl.BlockSpec((H, H), lambda b: (0, 0)),
            pl.BlockSpec((1, H), lambda b: (0, 0)),
            pl.BlockSpec((1, H), lambda b: (0, 0)),
            pl.BlockSpec((1, H), lambda b: (0, 0)),
            pl.BlockSpec((EC, 1, 1, C), lambda b: (0, b, 0, 0)),
            pl.BlockSpec((EC, 1, 1, C), lambda b: (0, b, 0, 0)),
        ],
        out_specs=pl.BlockSpec((1, 1, C), lambda b: (b, 0, 0)),
        out_shape=jax.ShapeDtypeStruct((B, 1, C), jnp.float32),
        scratch_shapes=[pltpu.VMEM((C, N, H), jnp.bfloat16),
                        pltpu.VMEM((N, C), jnp.float32)],
    )(h, agg, pn, gn_t, b7, wu, bu, g3, b3, s1, s0)


def kernel(x, edge_index, edge_attr, W_node, b_node, ln1_g, ln1_b, W_edge,
           b_edge, ln2_g, ln2_b, W_msg, b_msg, W_upd, b_upd, ln3_g, ln3_b,
           proto_n, proto_e, gate_n, gate_e):
    f32 = jnp.float32
    row = lambda v: v.reshape(1, H)
    h, hm = _run_node(x, W_node, row(b_node), row(ln1_g), row(ln1_b),
                      W_msg[:H, :])

    src = edge_index[0]
    dst = edge_index[1]
    srcg = (jnp.pad(src, (0, EP - E))[None, :]
            + (jnp.arange(B, dtype=jnp.int32) * N)[:, None]).reshape(-1)
    hms = _make_gather()(srcg, hm.reshape(B * N, H)).reshape(B, EP, H)

    b7 = jnp.repeat(jnp.eye(C, dtype=jnp.bfloat16), H, axis=0)
    pen, ges = _run_prep(proto_e, gate_e.T)
    m, s1, s0 = _run_edge(edge_attr, hms, pen, ges, b7, W_edge,
                          row(b_edge), row(ln2_g), row(ln2_b), W_msg[H:, :],
                          row(b_msg))

    dst_p = jnp.pad(dst, (0, EP - E))
    zeros = jnp.zeros((N, H), f32)
    agg2 = _make_scatter()(m.reshape(B * EP, H), dst_p, zeros)
    agg = agg2.reshape(B, N, H)

    logits = _run_final(h, agg, proto_n, gate_n.T, b7, W_upd, row(b_upd),
                        row(ln3_g), row(ln3_b), s1, s0)
    return logits.reshape(B, C)


# speedup vs baseline: 4.4582x; 1.4159x over previous
import functools

import jax
import jax.numpy as jnp
from jax import lax
from jax.experimental import pallas as pl
from jax.experimental.pallas import tpu as pltpu
from jax.experimental.pallas import tpu_sc as plsc

B = 16
C = 7
N = 4096
E = 32004
H = 64
EP = 32768
EBLK = 4096
EC = EP // EBLK
NC = 2
NS = 16
NW = NC * NS
GCH = 512
CPB = EP // GCH


def _ln(t, g, b):
    mu = jnp.mean(t, axis=-1, keepdims=True)
    d = t - mu
    v = jnp.mean(d * d, axis=-1, keepdims=True)
    return d * lax.rsqrt(v + 1e-5) * g + b


def _gelu(t):
    return 0.5 * t * (1.0 + lax.erf(t * 0.7071067811865476))


def _unit(t):
    n = jnp.sqrt(jnp.sum(t * t, axis=-1, keepdims=True))
    return t / jnp.maximum(n, 1e-6)


def _node_body(x_ref, wn_ref, bn_ref, g1_ref, b1_ref, wt_ref, h_ref, hm_ref):
    t = jnp.dot(x_ref[0], wn_ref[...], preferred_element_type=jnp.float32)
    t = t + bn_ref[...]
    h = _gelu(_ln(t, g1_ref[...], b1_ref[...]))
    h_ref[0] = h
    hm_ref[...] = jnp.dot(h, wt_ref[...], preferred_element_type=jnp.float32)


def _run_node(x_p, wn, bn, g1, b1, wt):
    return pl.pallas_call(
        _node_body,
        grid=(B,),
        in_specs=[
            pl.BlockSpec((1, N, 7), lambda b: (b, 0, 0)),
            pl.BlockSpec((7, H), lambda b: (0, 0)),
            pl.BlockSpec((1, H), lambda b: (0, 0)),
            pl.BlockSpec((1, H), lambda b: (0, 0)),
            pl.BlockSpec((1, H), lambda b: (0, 0)),
            pl.BlockSpec((H, H), lambda b: (0, 0)),
        ],
        out_specs=[
            pl.BlockSpec((1, N, H), lambda b: (b, 0, 0)),
            pl.BlockSpec((N, H), lambda b: (b, 0)),
        ],
        out_shape=[
            jax.ShapeDtypeStruct((B, N, H), jnp.float32),
            jax.ShapeDtypeStruct((B * N, H), jnp.float32),
        ],
    )(x_p, wn, bn, g1, b1, wt)


def _sc_mesh():
    return plsc.VectorSubcoreMesh(
        core_axis_name="c", subcore_axis_name="s",
        num_cores=NC, num_subcores=NS)


def _make_gather():
    @functools.partial(
        pl.kernel,
        mesh=_sc_mesh(),
        out_type=jax.ShapeDtypeStruct((B, EP, H), jnp.float32),
        scratch_types=[
            pltpu.VMEM((GCH,), jnp.int32),
            pltpu.VMEM((GCH,), jnp.int32),
            pltpu.VMEM((GCH, H), jnp.float32),
            pltpu.VMEM((GCH, H), jnp.float32),
            pltpu.SemaphoreType.DMA,
            pltpu.SemaphoreType.DMA,
            pltpu.SemaphoreType.DMA,
            pltpu.SemaphoreType.DMA,
        ],
        compiler_params=pltpu.CompilerParams(use_tc_tiling_on_sc=False),
    )
    def _gather_k(srcg_hbm, hm_hbm, out_hbm, i0, i1, r0, r1, gs0, gs1,
                  ws0, ws1):
        wid = lax.axis_index("s") * NC + lax.axis_index("c")
        n_chunks = (B * EP) // GCH // NW
        idx = [i0, i1]
        rows = [r0, r1]
        gs = [gs0, gs1]
        ws = [ws0, ws1]
        base = lambda i: (wid * n_chunks + i) * GCH
        pltpu.sync_copy(srcg_hbm.at[pl.ds(base(0), GCH)], i0)
        g = {0: pltpu.async_copy(hm_hbm.at[i0], r0, gs0)}
        w = {}
        for i in range(n_chunks):
            cur = i & 1
            nxt = 1 - cur
            if i + 1 < n_chunks:
                pltpu.sync_copy(srcg_hbm.at[pl.ds(base(i + 1), GCH)],
                                idx[nxt])
                if i >= 1:
                    w[i - 1].wait()
                g[i + 1] = pltpu.async_copy(hm_hbm.at[idx[nxt]], rows[nxt],
                                            gs[nxt])
            g[i].wait()
            gg = wid * n_chunks + i
            w[i] = pltpu.async_copy(
                rows[cur],
                out_hbm.at[gg // CPB, pl.ds((gg % CPB) * GCH, GCH)], ws[cur])
        w[n_chunks - 2].wait()
        w[n_chunks - 1].wait()

    return _gather_k


def _prep_body(pe_ref, gt_ref, pen_ref, ges_ref):
    j = pl.program_id(0)
    cidx = lax.broadcasted_iota(jnp.int32, (1, EBLK), 1) + j * EBLK
    colmask = cidx < E
    for c in range(C):
        pc = jnp.where(colmask, pe_ref[c], 0.0)
        ss = jnp.sum(pc * pc, axis=0, keepdims=True)
        pen_ref[c] = (pc * lax.rsqrt(jnp.maximum(ss, 1e-12))
                      ).astype(jnp.bfloat16)
    ges_ref[...] = jnp.where(colmask, jax.nn.sigmoid(gt_ref[...]), 0.0)


def _run_prep(pet, ge):
    return pl.pallas_call(
        _prep_body,
        grid=(EC,),
        in_specs=[
            pl.BlockSpec((C, H, EBLK), lambda j: (0, 0, j)),
            pl.BlockSpec((C, EBLK), lambda j: (0, j)),
        ],
        out_specs=[
            pl.BlockSpec((C, H, EBLK), lambda j: (0, 0, j)),
            pl.BlockSpec((C, EBLK), lambda j: (0, j)),
        ],
        out_shape=[
            jax.ShapeDtypeStruct((C, H, EP), jnp.bfloat16),
            jax.ShapeDtypeStruct((C, EP), jnp.float32),
        ],
    )(pet, ge)


def _ln_t(t, g, b):
    mu = jnp.mean(t, axis=0, keepdims=True)
    d = t - mu
    v = jnp.mean(d * d, axis=0, keepdims=True)
    return d * lax.rsqrt(v + 1e-5) * g + b


_LHS0 = (((0,), (0,)), ((), ()))


def _edge_body(ea_ref, hms_ref, pen_ref, ges_ref, b7t_ref, we_ref, be_ref,
               g2_ref, b2_ref, wb_ref, bm_ref, m_ref, s1_ref, s0_ref):
    j = pl.program_id(0)
    eab = ea_ref[0]
    tt = lax.dot_general(we_ref[...], eab, _LHS0,
                         preferred_element_type=jnp.float32)
    tt = tt + be_ref[...]
    et = _gelu(_ln_t(tt, g2_ref[...], b2_ref[...]))
    em = lax.dot_general(et, wb_ref[...], _LHS0,
                         preferred_element_type=jnp.float32)
    em = em + bm_ref[...]
    m = _gelu(hms_ref[0] + em)
    ridx = lax.broadcasted_iota(jnp.int32, (EBLK, 1), 0) + j * EBLK
    m_ref[0] = jnp.where(ridx < E, m, 0.0)

    ss = jnp.sum(et * et, axis=0, keepdims=True)
    ent = (et * lax.rsqrt(jnp.maximum(ss, 1e-12))).astype(jnp.bfloat16)
    ut = jnp.concatenate([ent * pen_ref[c] for c in range(C)], axis=0)
    simt = jnp.dot(b7t_ref[...], ut, preferred_element_type=jnp.float32)
    cidx = lax.broadcasted_iota(jnp.int32, (1, EBLK), 1) + j * EBLK
    simt = jnp.where(cidx < E, simt, 0.0)
    w = jax.nn.sigmoid(simt * 5.0) * ges_ref[...]
    s1_ref[...] = jnp.sum(w * simt, axis=1, keepdims=True).reshape(1, 1, C, 1)
    s0_ref[...] = jnp.sum(w, axis=1, keepdims=True).reshape(1, 1, C, 1)


def _run_edge(eat, hms, pen, ges, b7t, we, be, g2, b2, wb, bm):
    return pl.pallas_call(
        _edge_body,
        grid=(EC, B),
        in_specs=[
            pl.BlockSpec((1, 5, EBLK), lambda j, b: (b, 0, j)),
            pl.BlockSpec((1, EBLK, H), lambda j, b: (b, j, 0)),
            pl.BlockSpec((C, H, EBLK), lambda j, b: (0, 0, j)),
            pl.BlockSpec((C, EBLK), lambda j, b: (0, j)),
            pl.BlockSpec((C, C * H), lambda j, b: (0, 0)),
            pl.BlockSpec((5, H), lambda j, b: (0, 0)),
            pl.BlockSpec((H, 1), lambda j, b: (0, 0)),
            pl.BlockSpec((H, 1), lambda j, b: (0, 0)),
            pl.BlockSpec((H, 1), lambda j, b: (0, 0)),
            pl.BlockSpec((H, H), lambda j, b: (0, 0)),
            pl.BlockSpec((1, H), lambda j, b: (0, 0)),
        ],
        out_specs=[
            pl.BlockSpec((1, EBLK, H), lambda j, b: (b, j, 0)),
            pl.BlockSpec((1, 1, C, 1), lambda j, b: (j, b, 0, 0)),
            pl.BlockSpec((1, 1, C, 1), lambda j, b: (j, b, 0, 0)),
        ],
        out_shape=[
            jax.ShapeDtypeStruct((B, EP, H), jnp.float32),
            jax.ShapeDtypeStruct((EC, B, C, 1), jnp.float32),
            jax.ShapeDtypeStruct((EC, B, C, 1), jnp.float32),
        ],
    )(eat, hms, pen, ges, b7t, we, be, g2, b2, wb, bm)


EPT = EP // NS
BPC = B // NC
NPT = N // NS


SCH = 512
NSCH = EPT // SCH


def _make_scatter():
    @functools.partial(
        pl.kernel,
        mesh=_sc_mesh(),
        out_type=jax.ShapeDtypeStruct((B, N, H), jnp.float32),
        scratch_types=[
            pltpu.VMEM((SCH,), jnp.int32),
            pltpu.VMEM((SCH,), jnp.int32),
            pltpu.VMEM((SCH,), jnp.int32),
            pltpu.VMEM((SCH,), jnp.int32),
            pltpu.VMEM((SCH, H), jnp.float32),
            pltpu.VMEM((SCH, H), jnp.float32),
            pltpu.VMEM_SHARED((N, H), jnp.float32),
            pltpu.SemaphoreType.DMA,
            pltpu.SemaphoreType.DMA,
            pltpu.SemaphoreType.DMA,
            pltpu.SemaphoreType.DMA,
        ],
        compiler_params=pltpu.CompilerParams(use_tc_tiling_on_sc=False),
    )
    def _scatter_k(m_hbm, dst_hbm, zeros_hbm, out_hbm, ib0, ib1, ib2, ib3,
                   mb0, mb1, agg_s, ls0, ls1, ss0, ss1):
        c = lax.axis_index("c")
        s = lax.axis_index("s")
        ibs = [ib0, ib1, ib2, ib3]
        mb = [mb0, mb1]
        ls = [ls0, ls1]
        ssem = [ss0, ss1]
        for k in range(NSCH):
            pltpu.sync_copy(dst_hbm.at[pl.ds(s * EPT + k * SCH, SCH)], ibs[k])
        for t in range(BPC):
            b = c * BPC + t
            pltpu.sync_copy(zeros_hbm.at[pl.ds(s * NPT, NPT)],
                            agg_s.at[pl.ds(s * NPT, NPT)])
            plsc.subcore_barrier()
            mchunk = lambda k: m_hbm.at[b, pl.ds(s * EPT + k * SCH, SCH)]
            ld = {0: pltpu.async_copy(mchunk(0), mb0, ls0)}
            sc = {}
            for k in range(NSCH):
                p = k & 1
                q = 1 - p
                if k + 1 < NSCH:
                    if k >= 1:
                        sc[k - 1].wait()
                    ld[k + 1] = pltpu.async_copy(mchunk(k + 1), mb[q], ls[q])
                ld[k].wait()
                sc[k] = pltpu.async_copy(mb[p], agg_s.at[ibs[k]], ssem[p],
                                         add=True)
            sc[NSCH - 2].wait()
            sc[NSCH - 1].wait()
            plsc.subcore_barrier()
            pltpu.sync_copy(agg_s.at[pl.ds(s * NPT, NPT)],
                            out_hbm.at[b, pl.ds(s * NPT, NPT)])
            plsc.subcore_barrier()

    return _scatter_k


def _final_body(h_ref, agg_ref, pn_ref, gn_ref, b7_ref, wu_ref, bu_ref,
                g3_ref, b3_ref, s1_ref, s0_ref, out_ref, pnn_s, gn_s):
    ones64 = jnp.ones((H, H), jnp.bfloat16)

    @pl.when(pl.program_id(0) == 0)
    def _():
        for c in range(C):
            pc = pn_ref[c]
            ss = jnp.dot((pc * pc).astype(jnp.bfloat16), ones64,
                         preferred_element_type=jnp.float32)
            pnn_s[c] = (pc * lax.rsqrt(jnp.maximum(ss, 1e-12))
                        ).astype(jnp.bfloat16)
        gn_s[...] = jax.nn.sigmoid(gn_ref[...])

    a = jnp.dot(agg_ref[0], wu_ref[...], preferred_element_type=jnp.float32)
    a = a + bu_ref[...]
    h2 = _ln(h_ref[0] + a, g3_ref[...], b3_ref[...])
    ss = jnp.dot((h2 * h2).astype(jnp.bfloat16), ones64,
                 preferred_element_type=jnp.float32)
    hn = (h2 * lax.rsqrt(jnp.maximum(ss, 1e-12))).astype(jnp.bfloat16)
    u = jnp.concatenate([hn * pnn_s[c] for c in range(C)], axis=-1)
    sim = jnp.dot(u, b7_ref[...], preferred_element_type=jnp.float32)
    w = jax.nn.sigmoid(sim * 5.0) * gn_s[...]
    ns = jnp.sum(w * sim, axis=0) / jnp.maximum(jnp.sum(w, axis=0), 1e-6)
    es1 = jnp.sum(s1_ref[...], axis=(0, 1, 3))
    es0 = jnp.sum(s0_ref[...], axis=(0, 1, 3))
    es = es1 / jnp.maximum(es0, 1e-6)
    out_ref[...] = (ns + 0.5 * es).reshape(1, 1, C)


def _run_final(h, agg, pn, gn_t, b7, wu, bu, g3, b3, s1, s0):
    return pl.pallas_call(
        _final_body,
        grid=(B,),
        in_specs=[
            pl.BlockSpec((1, N, H), lambda b: (b, 0, 0)),
            pl.BlockSpec((1, N, H), lambda b: (b, 0, 0)),
            pl.BlockSpec((C, N, H), lambda b: (0, 0, 0)),
            pl.BlockSpec((N, C), lambda b: (0, 0)),
            pl.BlockSpec((C * H, C), lambda b: (0, 0)),
            pl.BlockSpec((H, H), lambda b: (0, 0)),
            pl.BlockSpec((1, H), lambda b: (0, 0)),
            pl.BlockSpec((1, H), lambda b: (0, 0)),
            pl.BlockSpec((1, H), lambda b: (0, 0)),
            pl.BlockSpec((EC, 1, C, 1), lambda b: (0, b, 0, 0)),
            pl.BlockSpec((EC, 1, C, 1), lambda b: (0, b, 0, 0)),
        ],
        out_specs=pl.BlockSpec((1, 1, C), lambda b: (b, 0, 0)),
        out_shape=jax.ShapeDtypeStruct((B, 1, C), jnp.float32),
        scratch_shapes=[pltpu.VMEM((C, N, H), jnp.bfloat16),
                        pltpu.VMEM((N, C), jnp.float32)],
    )(h, agg, pn, gn_t, b7, wu, bu, g3, b3, s1, s0)


def kernel(x, edge_index, edge_attr, W_node, b_node, ln1_g, ln1_b, W_edge,
           b_edge, ln2_g, ln2_b, W_msg, b_msg, W_upd, b_upd, ln3_g, ln3_b,
           proto_n, proto_e, gate_n, gate_e):
    f32 = jnp.float32
    row = lambda v: v.reshape(1, H)
    h, hm = _run_node(x, W_node, row(b_node), row(ln1_g), row(ln1_b),
                      W_msg[:H, :])

    src = edge_index[0]
    dst = edge_index[1]
    srcg = (jnp.pad(src, (0, EP - E))[None, :]
            + (jnp.arange(B, dtype=jnp.int32) * N)[:, None]).reshape(-1)
    hms = _make_gather()(srcg, hm)

    eat = jnp.transpose(edge_attr, (0, 2, 1))
    pet = jnp.transpose(proto_e, (0, 2, 1))
    b7t = jnp.repeat(jnp.eye(C, dtype=jnp.bfloat16), H, axis=1)
    b7 = jnp.repeat(jnp.eye(C, dtype=jnp.bfloat16), H, axis=0)
    col = lambda v: v.reshape(H, 1)
    pen, ges = _run_prep(pet, gate_e)
    m, s1, s0 = _run_edge(eat, hms, pen, ges, b7t, W_edge,
                          col(b_edge), col(ln2_g), col(ln2_b), W_msg[H:, :],
                          row(b_msg))

    dst_p = jnp.pad(dst, (0, EP - E))
    zeros = jnp.zeros((N, H), f32)
    agg = _make_scatter()(m, dst_p, zeros)

    logits = _run_final(h, agg, proto_n, gate_n.T, b7, W_upd, row(b_upd),
                        row(ln3_g), row(ln3_b), s1, s0)
    return logits.reshape(B, C)
